# baseline (speedup 1.0000x reference)
import jax
import jax.numpy as jnp
from jax import lax
from jax.experimental import pallas as pl
from jax.experimental.pallas import tpu as pltpu
from jax.experimental.pallas import tpu_sc as plsc

_NUM_CORES = 2
_NUM_SUBCORES = 16
_NUM_WORKERS = _NUM_CORES * _NUM_SUBCORES

_BATCH = 64
_SEQ = 1024
_HIDDEN = 512
_TOTAL = _BATCH * _SEQ
_BPW = _TOTAL // _NUM_WORKERS
_CHUNK = 64
_NCHUNK = _BPW // _CHUNK


def _body(x_hbm, tab_hbm, out_hbm, idx_v, rows0, rows1, sem0, sem1):
    wid = lax.axis_index("s") * _NUM_CORES + lax.axis_index("c")
    base = wid * _BPW
    pltpu.sync_copy(x_hbm.at[pl.ds(base, _BPW)], idx_v)

    rows = (rows0, rows1)
    sems = (sem0, sem1)
    copies = [None, None]
    copies[0] = pltpu.async_copy(
        tab_hbm.at[idx_v.at[pl.ds(0, _CHUNK)]], rows[0], sems[0])
    for g in range(_NCHUNK):
        nxt = g + 1
        if nxt < _NCHUNK:
            copies[nxt % 2] = pltpu.async_copy(
                tab_hbm.at[idx_v.at[pl.ds(nxt * _CHUNK, _CHUNK)]],
                rows[nxt % 2], sems[nxt % 2])
        copies[g % 2].wait()
        pltpu.sync_copy(rows[g % 2],
                        out_hbm.at[pl.ds(base + g * _CHUNK, _CHUNK)])


@jax.jit
def _lookup(x_flat, token_embed):
    mesh = plsc.VectorSubcoreMesh(core_axis_name="c", subcore_axis_name="s")
    run = pl.kernel(
        _body,
        out_type=jax.ShapeDtypeStruct((_TOTAL, _HIDDEN), jnp.float32),
        mesh=mesh,
        scratch_types=[
            pltpu.VMEM((_BPW,), jnp.int32),
            pltpu.VMEM((_CHUNK, _HIDDEN), jnp.float32),
            pltpu.VMEM((_CHUNK, _HIDDEN), jnp.float32),
            pltpu.SemaphoreType.DMA,
            pltpu.SemaphoreType.DMA,
        ],
    )
    return run(x_flat, token_embed)


def kernel(x, token_embed):
    out = _lookup(x.reshape(_TOTAL).astype(jnp.int32), token_embed)
    return out.reshape(_BATCH, _SEQ, _HIDDEN)
